# Initial kernel scaffold; baseline (speedup 1.0000x reference)
import jax
import jax.numpy as jnp
from jax.experimental import pallas as pl


def kernel(coords, tables):
    raise NotImplementedError("write your pallas kernel here")


# speedup vs baseline: 9.7627x; 9.7627x over previous
import functools

import numpy as np
import jax
import jax.numpy as jnp
from jax import lax
from jax._src.config import enable_x64 as _enable_x64
from jax.experimental import pallas as pl
from jax.experimental.pallas import tpu as pltpu
from jax.experimental.pallas import tpu_sc as plsc

NUM_LEVELS = 16
LOG2_HASHMAP = 19
HASHMAP_SIZE = 1 << LOG2_HASHMAP
MASK = HASHMAP_SIZE - 1
N_POINTS = 1048576
BASE_RES = 16
MAX_RES = 512
RES = [int(BASE_RES * (MAX_RES / BASE_RES) ** (i / (NUM_LEVELS - 1)))
       for i in range(NUM_LEVELS)]
P1 = np.uint32(2654435761).view(np.int32)
P2 = np.int32(805459861)

NC, NS = 2, 16
NW = NC * NS
PPW = N_POINTS // NW
C = 128
WPP = NUM_LEVELS * 2
WPC = C * WPP
G = WPC // 128
CHUNKS = PPW // C

_mesh = plsc.VectorSubcoreMesh(core_axis_name="c", subcore_axis_name="s")


@functools.partial(
    pl.kernel,
    out_type=jax.ShapeDtypeStruct((N_POINTS * WPP,), jnp.float32),
    mesh=_mesh,
    scratch_types=[
        pltpu.VMEM((C * 3,), jnp.float32),
        pltpu.VMEM((G, 128), jnp.int32),
        pltpu.VMEM((WPC,), jnp.float32),
        pltpu.SemaphoreType.DMA,
    ],
    compiler_params=pltpu.CompilerParams(
        needs_layout_passes=False, use_tc_tiling_on_sc=False
    ),
)
def _sc_encode(coords_hbm, table_hbm, out_hbm, cv, idxv, vals, sem):
    wid = lax.axis_index("s") * np.int32(NC) + lax.axis_index("c")
    iota = lax.iota(jnp.int32, 16)
    iota3 = iota * np.int32(3)
    row_pat = lax.shift_right_logical(iota, np.int32(2))
    col_pat = (iota & np.int32(3)) * np.int32(32)

    def chunk_body(ci, _):
        base = wid * np.int32(PPW) + ci * np.int32(C)
        pltpu.sync_copy(coords_hbm.at[pl.ds(base * np.int32(3), C * 3)], cv)

        def group_body(i, _):
            off = i * np.int32(48) + iota3
            xs = plsc.load_gather(cv, [off])
            ys = plsc.load_gather(cv, [off + np.int32(1)])
            zs = plsc.load_gather(cv, [off + np.int32(2)])
            rowi = i * np.int32(4) + row_pat
            for lvl in range(NUM_LEVELS):
                r = np.float32(RES[lvl])
                gx = (xs * r).astype(jnp.int32)
                gy = (ys * r).astype(jnp.int32)
                gz = (zs * r).astype(jnp.int32)
                h = (gx ^ (gy * P1) ^ (gz * P2)) & np.int32(MASK)
                w = h + h + np.int32(lvl * HASHMAP_SIZE * 2)
                coli = col_pat + np.int32(2 * lvl)
                plsc.store_scatter(idxv, [rowi, coli], w)
                plsc.store_scatter(idxv, [rowi, coli + np.int32(1)], w + np.int32(1))
            return 0

        lax.fori_loop(np.int32(0), np.int32(C // 16), group_body, 0)

        def fire(j, _):
            pltpu.async_copy(
                table_hbm.at[idxv.at[j]],
                vals.at[pl.ds(j * np.int32(128), 128)],
                sem,
            )
            return 0

        lax.fori_loop(np.int32(0), np.int32(G), fire, 0)

        def drain(j, _):
            pltpu.make_async_copy(
                table_hbm.at[idxv.at[j]],
                vals.at[pl.ds(j * np.int32(128), 128)],
                sem,
            ).wait()
            return 0

        lax.fori_loop(np.int32(0), np.int32(G), drain, 0)
        pltpu.sync_copy(vals, out_hbm.at[pl.ds(base * np.int32(WPP), WPC)])
        return 0

    lax.fori_loop(np.int32(0), np.int32(CHUNKS), chunk_body, 0)


def kernel(coords, tables):
    with _enable_x64(False):
        coords_flat = coords.reshape(N_POINTS * 3)
        table_flat = tables.reshape(NUM_LEVELS * HASHMAP_SIZE * 2)
        out = _sc_encode(coords_flat, table_flat)
        return out.reshape(N_POINTS, WPP)
